# Initial kernel scaffold; baseline (speedup 1.0000x reference)
import jax
import jax.numpy as jnp
from jax.experimental import pallas as pl


def kernel(features, receptivity, gate_W1, gate_b1, gate_W2, gate_b2, exp_W1, exp_b1, exp_W2, exp_b2):
    raise NotImplementedError("write your pallas kernel here")


# speedup vs baseline: 2.1656x; 2.1656x over previous
import functools

import jax
import jax.numpy as jnp
from jax.experimental import pallas as pl
from jax.experimental.pallas import tpu as pltpu

E = 8
TOP_K = 2
D_IN = 1024
D_HID = 512
D_OUT = 1024
N = 2048
TILE = 256
NT = N // TILE

_NEG_INF = float("-inf")


def _gate_body(feat_ref, gw1_ref, gw2_ref, gb1_ref, gb2_ref, rec_ref,
               tki_ref, tkw_ref, cw_ref, ranks_ref, totals_ref, gh_acc):
    i = pl.program_id(0)
    e = pl.program_id(1)
    x = feat_ref[0]
    w1 = gw1_ref[0]
    part = jnp.dot(x, w1, preferred_element_type=jnp.float32)

    @pl.when(e == 0)
    def _():
        gh_acc[...] = part

    @pl.when(e > 0)
    def _():
        gh_acc[...] = gh_acc[...] + part

    @pl.when(e == E - 1)
    def _():
        gh = jax.nn.relu(gh_acc[...] + gb1_ref[...])
        logits = jnp.dot(gh, gw2_ref[...],
                         preferred_element_type=jnp.float32) + gb2_ref[...]
        m = jnp.max(logits, axis=1, keepdims=True)
        ex = jnp.exp(logits - m)
        sm = ex / jnp.sum(ex, axis=1, keepdims=True)
        scores = sm + rec_ref[...]

        iota = jax.lax.broadcasted_iota(jnp.int32, (TILE, E), 1)
        v1 = jnp.max(scores, axis=1, keepdims=True)
        i1 = jnp.min(jnp.where(scores == v1, iota, E), axis=1, keepdims=True)
        masked = jnp.where(iota == i1, _NEG_INF, scores)
        v2 = jnp.max(masked, axis=1, keepdims=True)
        i2 = jnp.min(jnp.where(masked == v2, iota, E), axis=1, keepdims=True)
        s = v1 + v2
        w_1 = v1 / s
        w_2 = v2 / s

        tki_ref[...] = jnp.concatenate([i1, i2], axis=1)
        tkw_ref[...] = jnp.concatenate([w_1, w_2], axis=1)
        oh1 = (iota == i1)
        oh2 = (iota == i2)
        cw_ref[...] = (jnp.where(oh1, w_1, 0.0)
                       + jnp.where(oh2, w_2, 0.0))
        r = 2 - 2 * oh1.astype(jnp.int32) - oh2.astype(jnp.int32)
        ranks_ref[...] = r
        part_tot = jnp.sum(r, axis=0, keepdims=True)

        @pl.when(i == 0)
        def _():
            totals_ref[...] = part_tot

        @pl.when(i > 0)
        def _():
            totals_ref[...] = totals_ref[...] + part_tot


def _expert_body(feat_ref, w1_ref, b1_ref, w2_ref, b2_ref, cw_ref,
                 out_ref, acc):
    e = pl.program_id(1)
    x = feat_ref[0]
    h = jax.nn.relu(jnp.dot(x, w1_ref[0], preferred_element_type=jnp.float32)
                    + b1_ref[0])
    o = jnp.dot(h, w2_ref[0], preferred_element_type=jnp.float32) + b2_ref[0]
    emask = (jax.lax.broadcasted_iota(jnp.int32, (1, E), 1) == e)
    col = jnp.sum(jnp.where(emask, cw_ref[...], 0.0), axis=1, keepdims=True)
    contrib = col * o

    @pl.when(e == 0)
    def _():
        acc[...] = contrib

    @pl.when(e > 0)
    def _():
        acc[...] = acc[...] + contrib

    @pl.when(e == E - 1)
    def _():
        out_ref[...] = acc[...]


@jax.jit
def kernel(features, receptivity, gate_W1, gate_b1, gate_W2, gate_b2,
           exp_W1, exp_b1, exp_W2, exp_b2):
    gw1 = gate_W1.reshape(E, D_IN, D_HID)
    gb1 = gate_b1.reshape(1, D_HID)
    gb2 = gate_b2.reshape(1, E)
    rec = jnp.transpose(receptivity[..., 0], (1, 0))
    eb1 = exp_b1.reshape(E, 1, D_HID)
    eb2 = exp_b2.reshape(E, 1, D_OUT)

    tki, tkw, cw, ranks_t, totals = pl.pallas_call(
        _gate_body,
        grid=(NT, E),
        in_specs=[
            pl.BlockSpec((1, TILE, D_IN), lambda i, e: (e, i, 0)),
            pl.BlockSpec((1, D_IN, D_HID), lambda i, e: (e, 0, 0)),
            pl.BlockSpec((D_HID, E), lambda i, e: (0, 0)),
            pl.BlockSpec((1, D_HID), lambda i, e: (0, 0)),
            pl.BlockSpec((1, E), lambda i, e: (0, 0)),
            pl.BlockSpec((TILE, E), lambda i, e: (i, 0)),
        ],
        out_specs=[
            pl.BlockSpec((TILE, TOP_K), lambda i, e: (i, 0)),
            pl.BlockSpec((TILE, TOP_K), lambda i, e: (i, 0)),
            pl.BlockSpec((TILE, E), lambda i, e: (i, 0)),
            pl.BlockSpec((TILE, E), lambda i, e: (i, 0)),
            pl.BlockSpec((1, E), lambda i, e: (0, 0)),
        ],
        out_shape=[
            jax.ShapeDtypeStruct((N, TOP_K), jnp.int32),
            jax.ShapeDtypeStruct((N, TOP_K), jnp.float32),
            jax.ShapeDtypeStruct((N, E), jnp.float32),
            jax.ShapeDtypeStruct((N, E), jnp.int32),
            jax.ShapeDtypeStruct((1, E), jnp.int32),
        ],
        scratch_shapes=[pltpu.VMEM((TILE, D_HID), jnp.float32)],
    )(features, gw1, gate_W2, gb1, gb2, rec)

    final_out = pl.pallas_call(
        _expert_body,
        grid=(NT, E),
        in_specs=[
            pl.BlockSpec((1, TILE, D_IN), lambda i, e: (e, i, 0)),
            pl.BlockSpec((1, D_IN, D_HID), lambda i, e: (e, 0, 0)),
            pl.BlockSpec((1, 1, D_HID), lambda i, e: (e, 0, 0)),
            pl.BlockSpec((1, D_HID, D_OUT), lambda i, e: (e, 0, 0)),
            pl.BlockSpec((1, 1, D_OUT), lambda i, e: (e, 0, 0)),
            pl.BlockSpec((TILE, E), lambda i, e: (i, 0)),
        ],
        out_specs=pl.BlockSpec((TILE, D_OUT), lambda i, e: (i, 0)),
        out_shape=jax.ShapeDtypeStruct((N, D_OUT), jnp.float32),
        scratch_shapes=[pltpu.VMEM((TILE, D_OUT), jnp.float32)],
    )(features, exp_W1, eb1, exp_W2, eb2, cw)

    del tki, tkw
    return final_out, jnp.transpose(ranks_t, (1, 0)), totals.reshape(E)
